# baseline (speedup 1.0000x reference)
import functools

import jax
import jax.numpy as jnp
from jax import lax
from jax.experimental import pallas as pl
from jax.experimental.pallas import tpu as pltpu
from jax.experimental.pallas import tpu_sc as plsc

N = 1024 * 26 * 20
DIM = 32
VOCAB = 1000000
L = 16
NC, NS = 2, 16
NW = NC * NS
CH = N // NW
ROWS = N // 128
CR = ROWS // NW
HR = ROWS // NC
N2 = N // NC
NP = HR // 32
VP = 1000448
SHH = VP // NS
SENT = 2**30

_mesh = plsc.VectorSubcoreMesh(
    core_axis_name="c", subcore_axis_name="s", num_cores=NC, num_subcores=NS)
_params = pltpu.CompilerParams(
    needs_layout_passes=False, use_tc_tiling_on_sc=False)


def _wid():
  return lax.axis_index("s") * NC + lax.axis_index("c")


def _iota16():
  return lax.iota(jnp.int32, 16)


def _ka(idr_hbm, ta_hbm, tb_hbm, tbl, ib0, ib1, s0, s1):
  sid = lax.axis_index("s")
  cid = lax.axis_index("c")
  base = sid * SHH
  hi = base + SHH

  sent = jnp.full((L,), SENT, jnp.int32)

  def init_body(i, _):
    i = i.astype(jnp.int32)
    tbl[pl.ds(i * L, L)] = sent
    return 0

  lax.fori_loop(0, SHH // L, init_body, 0, unroll=4)

  iot = _iota16()
  row0 = cid * HR

  def start(j, buf, sem):
    pltpu.make_async_copy(
        idr_hbm.at[pl.ds(row0 + j * 32, 32), :], buf, sem).start()

  def wait(buf, sem):
    pltpu.make_async_copy(idr_hbm.at[pl.ds(0, 32), :], buf, sem).wait()

  def process(buf, j):
    pbase = cid * N2 + j * 4096

    def row(rr, _):
      r = 31 - rr.astype(jnp.int32)
      lms, idxs, rps = [], [], []
      for c in range(7, -1, -1):
        rid = lax.rev(buf[r, pl.ds(c * L, L)], (0,))
        m = (rid >= base) & (rid < hi)
        _, lastm = plsc.scan_count(rid, mask=m)
        lms.append(lastm & m)
        idxs.append(jnp.where(m, rid - base, 0))
        rps.append((pbase + r * 128 + c * L + 15) - iot)
      for k in range(8):
        plsc.store_scatter(tbl, [idxs[k]], rps[k], mask=lms[k])
      return 0

    lax.fori_loop(0, 32, row, 0, unroll=2)

  start(NP - 1, ib0, s0)

  def piece(g, _):
    g = g.astype(jnp.int32)
    j0 = (NP - 1) - 2 * g

    @pl.when(j0 >= 1)
    def _():
      start(j0 - 1, ib1, s1)

    wait(ib0, s0)
    process(ib0, j0)

    j1 = j0 - 1

    @pl.when(j1 >= 0)
    def _():
      @pl.when(j1 >= 1)
      def _():
        start(j1 - 1, ib0, s0)

      wait(ib1, s1)
      process(ib1, j1)

    return 0

  lax.fori_loop(0, (NP + 1) // 2, piece, 0)

  @pl.when(cid == 0)
  def _():
    pltpu.sync_copy(tbl, ta_hbm.at[pl.ds(base, SHH)])

  @pl.when(cid == 1)
  def _():
    pltpu.sync_copy(tbl, tb_hbm.at[pl.ds(base, SHH)])


_phase_a = functools.partial(
    pl.kernel,
    out_type=(
        jax.ShapeDtypeStruct((VP,), jnp.int32),
        jax.ShapeDtypeStruct((VP,), jnp.int32),
    ),
    mesh=_mesh,
    compiler_params=_params,
    scratch_types=[
        pltpu.VMEM((SHH,), jnp.int32),
        pltpu.VMEM((32, 128), jnp.int32),
        pltpu.VMEM((32, 128), jnp.int32),
        pltpu.SemaphoreType.DMA,
        pltpu.SemaphoreType.DMA,
    ],
)(_ka)


def _kb(ids_hbm, ta_hbm, tb_hbm, fpo_hbm, s_hbm, part_hbm,
        idv, fpva, fpvb, sbuf, pbuf, sa, sb):
  wid = _wid()
  rbase = wid * CR
  pbase = rbase * 128

  pltpu.sync_copy(ids_hbm.at[pl.ds(rbase, CR), :], idv)

  iot = _iota16()
  GR = 5

  def issue(g):
    for b in range(GR):
      j = g * GR + b
      pltpu.make_async_copy(ta_hbm.at[idv.at[j]], fpva.at[j], sa).start()
      pltpu.make_async_copy(tb_hbm.at[idv.at[j]], fpvb.at[j], sb).start()

  def drain():
    for b in range(GR):
      pltpu.make_async_copy(ta_hbm.at[idv.at[0]], fpva.at[0], sa).wait()
      pltpu.make_async_copy(tb_hbm.at[idv.at[0]], fpvb.at[0], sb).wait()

  def crow(r, carry):
    exs, tots = [], []
    for c in range(8):
      v = jnp.minimum(fpva[r, pl.ds(c * L, L)], fpvb[r, pl.ds(c * L, L)])
      fpva[r, pl.ds(c * L, L)] = v
      pos = (pbase + r * 128 + c * L) + iot
      isf = jnp.where(v == pos, 1, 0).astype(jnp.int32)
      exs.append(plsc.cumsum(isf) - isf)
      tots.append(jnp.sum(isf, dtype=jnp.int32))
    for c in range(8):
      sbuf[pl.ds(r * 128 + c * L, L)] = exs[c] + carry
      carry = carry + tots[c]
    return carry

  issue(jnp.int32(0))

  def grp(g, carry):
    g = g.astype(jnp.int32)

    @pl.when(g + 1 < CR // GR)
    def _():
      issue(g + 1)

    drain()
    for b in range(GR):
      carry = crow(g * GR + b, carry)
    return carry

  total = lax.fori_loop(0, CR // GR, grp, jnp.int32(0))

  pltpu.sync_copy(fpva, fpo_hbm.at[pl.ds(rbase, CR), :])
  pltpu.sync_copy(sbuf, s_hbm.at[pl.ds(pbase, CH)])
  pbuf[...] = jnp.full((L,), 0, jnp.int32) + total
  pltpu.sync_copy(pbuf, part_hbm.at[wid])


_phase_b = functools.partial(
    pl.kernel,
    out_type=(
        jax.ShapeDtypeStruct((ROWS, 128), jnp.int32),
        jax.ShapeDtypeStruct((N,), jnp.int32),
        jax.ShapeDtypeStruct((NW, L), jnp.int32),
    ),
    mesh=_mesh,
    compiler_params=_params,
    scratch_types=[
        pltpu.VMEM((CR, 128), jnp.int32),
        pltpu.VMEM((CR, 128), jnp.int32),
        pltpu.VMEM((CR, 128), jnp.int32),
        pltpu.VMEM((CH,), jnp.int32),
        pltpu.VMEM((L,), jnp.int32),
        pltpu.SemaphoreType.DMA,
        pltpu.SemaphoreType.DMA,
    ],
)(_kb)


def _kc(fpo_hbm, s_hbm, part_hbm, emb_hbm, out_hbm,
        pv, offv, fpv, sv, rb0, rb1, rb2, rb3, rb4, rb5, rb6, rb7,
        sem, sg0, sg1, sg2, sg3, sg4, sg5, sg6, sg7,
        sw0, sw1, sw2, sw3, sw4, sw5, sw6, sw7):
  wid = _wid()
  rbase = wid * CR

  pltpu.sync_copy(part_hbm, pv)
  iot = _iota16()
  z = jnp.zeros((L,), jnp.int32)
  t0 = plsc.load_gather(pv, [iot, z])
  t1 = plsc.load_gather(pv, [iot + 16, z])
  c0 = plsc.cumsum(t0)
  c1 = plsc.cumsum(t1) + jnp.sum(t0, dtype=jnp.int32)
  offv[pl.ds(0, L)] = c0 - t0
  offv[pl.ds(L, L)] = c1 - t1

  pltpu.sync_copy(fpo_hbm.at[pl.ds(rbase, CR), :], fpv)

  GR = 5

  def issue(g):
    for b in range(GR):
      j = g * GR + b
      pltpu.make_async_copy(s_hbm.at[fpv.at[j]], sv.at[j], sem).start()

  def drain():
    for b in range(GR):
      pltpu.make_async_copy(s_hbm.at[fpv.at[0]], sv.at[0], sem).wait()

  def hrow(r):
    for c in range(8):
      f = fpv[r, pl.ds(c * L, L)]
      s = sv[r, pl.ds(c * L, L)]
      o = plsc.load_gather(offv, [f // CH])
      sv[r, pl.ds(c * L, L)] = s + o

  issue(jnp.int32(0))

  def hgrp(g, _):
    g = g.astype(jnp.int32)

    @pl.when(g + 1 < CR // GR)
    def _():
      issue(g + 1)

    drain()
    for b in range(GR):
      hrow(g * GR + b)
    return 0

  lax.fori_loop(0, CR // GR, hgrp, 0)

  obase = wid * CH
  rbs = (rb0, rb1, rb2, rb3, rb4, rb5, rb6, rb7)
  sgs = (sg0, sg1, sg2, sg3, sg4, sg5, sg6, sg7)
  sws = (sw0, sw1, sw2, sw3, sw4, sw5, sw6, sw7)

  def gst(j, b):
    pltpu.make_async_copy(emb_hbm.at[sv.at[j]], rbs[b], sgs[b]).start()

  def gwt(b):
    pltpu.make_async_copy(emb_hbm.at[sv.at[0]], rbs[b], sgs[b]).wait()

  def wst(j, b):
    pltpu.make_async_copy(
        rbs[b], out_hbm.at[pl.ds(obase + j * 128, 128), :], sws[b]).start()

  def wwt(b):
    pltpu.make_async_copy(
        rbs[b], out_hbm.at[pl.ds(obase, 128), :], sws[b]).wait()

  for b in range(8):
    gst(b, b)

  def grow(g, _):
    j0 = 8 * g.astype(jnp.int32)
    for b in range(8):
      j = j0 + b

      @pl.when(j < CR)
      def _():
        gwt(b)
        wst(j, b)

    for b in range(8):
      jn = j0 + 8 + b

      @pl.when(jn < CR)
      def _():
        wwt(b)
        gst(jn, b)

    return 0

  lax.fori_loop(0, (CR + 7) // 8, grow, 0)

  for b in range(8):
    wwt(b)


_phase_c = functools.partial(
    pl.kernel,
    out_type=jax.ShapeDtypeStruct((N, DIM), jnp.float32),
    mesh=_mesh,
    compiler_params=_params,
    scratch_types=[
        pltpu.VMEM((NW, L), jnp.int32),
        pltpu.VMEM((NW,), jnp.int32),
        pltpu.VMEM((CR, 128), jnp.int32),
        pltpu.VMEM((CR, 128), jnp.int32),
        pltpu.VMEM((128, DIM), jnp.float32),
        pltpu.VMEM((128, DIM), jnp.float32),
        pltpu.VMEM((128, DIM), jnp.float32),
        pltpu.VMEM((128, DIM), jnp.float32),
        pltpu.VMEM((128, DIM), jnp.float32),
        pltpu.VMEM((128, DIM), jnp.float32),
        pltpu.VMEM((128, DIM), jnp.float32),
        pltpu.VMEM((128, DIM), jnp.float32),
        pltpu.SemaphoreType.DMA,
        pltpu.SemaphoreType.DMA,
        pltpu.SemaphoreType.DMA,
        pltpu.SemaphoreType.DMA,
        pltpu.SemaphoreType.DMA,
        pltpu.SemaphoreType.DMA,
        pltpu.SemaphoreType.DMA,
        pltpu.SemaphoreType.DMA,
        pltpu.SemaphoreType.DMA,
        pltpu.SemaphoreType.DMA,
        pltpu.SemaphoreType.DMA,
        pltpu.SemaphoreType.DMA,
        pltpu.SemaphoreType.DMA,
        pltpu.SemaphoreType.DMA,
        pltpu.SemaphoreType.DMA,
        pltpu.SemaphoreType.DMA,
        pltpu.SemaphoreType.DMA,
    ],
)(_kc)


def kernel(ids, embedding_var, default_embedding):
  del default_embedding
  with jax.enable_x64(False):
    ids32 = ids.astype(jnp.int32).reshape(ROWS, 128)
    emb = embedding_var[:N]
    ta, tb = _phase_a(ids32)
    fpo, s_local, partials = _phase_b(ids32, ta, tb)
    out = _phase_c(fpo, s_local, partials, emb)
  return out.reshape(ids.shape + (DIM,))


# speedup vs baseline: 1.0504x; 1.0504x over previous
import functools

import jax
import jax.numpy as jnp
from jax import lax
from jax.experimental import pallas as pl
from jax.experimental.pallas import tpu as pltpu
from jax.experimental.pallas import tpu_sc as plsc

N = 1024 * 26 * 20
DIM = 32
VOCAB = 1000000
L = 16
NC, NS = 2, 16
NW = NC * NS
CH = N // NW
ROWS = N // 128
CR = ROWS // NW
HR = ROWS // NC
N2 = N // NC
NP = HR // 32
VP = 1000448
SHH = VP // NS
SENT = 2**30

_mesh = plsc.VectorSubcoreMesh(
    core_axis_name="c", subcore_axis_name="s", num_cores=NC, num_subcores=NS)
_params = pltpu.CompilerParams(
    needs_layout_passes=False, use_tc_tiling_on_sc=False)


def _wid():
  return lax.axis_index("s") * NC + lax.axis_index("c")


def _iota16():
  return lax.iota(jnp.int32, 16)


def _ka(idr_hbm, ta_hbm, tb_hbm, tbl, ib0, ib1, s0, s1):
  sid = lax.axis_index("s")
  cid = lax.axis_index("c")
  base = sid * SHH
  hi = base + SHH

  sent = jnp.full((L,), SENT, jnp.int32)

  def init_body(i, _):
    i = i.astype(jnp.int32)
    tbl[pl.ds(i * L, L)] = sent
    return 0

  lax.fori_loop(0, SHH // L, init_body, 0, unroll=4)

  iot = _iota16()
  row0 = cid * HR

  def start(j, buf, sem):
    pltpu.make_async_copy(
        idr_hbm.at[pl.ds(row0 + j * 32, 32), :], buf, sem).start()

  def wait(buf, sem):
    pltpu.make_async_copy(idr_hbm.at[pl.ds(0, 32), :], buf, sem).wait()

  def process(buf, j):
    pbase = cid * N2 + j * 4096

    def row(rr, _):
      r = 31 - rr.astype(jnp.int32)
      lms, idxs, rps = [], [], []
      for c in range(7, -1, -1):
        rid = lax.rev(buf[r, pl.ds(c * L, L)], (0,))
        m = (rid >= base) & (rid < hi)
        _, lastm = plsc.scan_count(rid, mask=m)
        lms.append(lastm & m)
        idxs.append(jnp.where(m, rid - base, 0))
        rps.append((pbase + r * 128 + c * L + 15) - iot)
      for k in range(8):
        plsc.store_scatter(tbl, [idxs[k]], rps[k], mask=lms[k])
      return 0

    lax.fori_loop(0, 32, row, 0, unroll=2)

  start(NP - 1, ib0, s0)

  def piece(g, _):
    g = g.astype(jnp.int32)
    j0 = (NP - 1) - 2 * g

    @pl.when(j0 >= 1)
    def _():
      start(j0 - 1, ib1, s1)

    wait(ib0, s0)
    process(ib0, j0)

    j1 = j0 - 1

    @pl.when(j1 >= 0)
    def _():
      @pl.when(j1 >= 1)
      def _():
        start(j1 - 1, ib0, s0)

      wait(ib1, s1)
      process(ib1, j1)

    return 0

  lax.fori_loop(0, (NP + 1) // 2, piece, 0)

  @pl.when(cid == 0)
  def _():
    pltpu.sync_copy(tbl, ta_hbm.at[pl.ds(base, SHH)])

  @pl.when(cid == 1)
  def _():
    pltpu.sync_copy(tbl, tb_hbm.at[pl.ds(base, SHH)])


_phase_a = functools.partial(
    pl.kernel,
    out_type=(
        jax.ShapeDtypeStruct((VP,), jnp.int32),
        jax.ShapeDtypeStruct((VP,), jnp.int32),
    ),
    mesh=_mesh,
    compiler_params=_params,
    scratch_types=[
        pltpu.VMEM((SHH,), jnp.int32),
        pltpu.VMEM((32, 128), jnp.int32),
        pltpu.VMEM((32, 128), jnp.int32),
        pltpu.SemaphoreType.DMA,
        pltpu.SemaphoreType.DMA,
    ],
)(_ka)


def _kb(ids_hbm, ta_hbm, tb_hbm, fpo_hbm, s_hbm, part_hbm,
        idv, fpva, fpvb, sbuf, pbuf, sa, sb):
  wid = _wid()
  rbase = wid * CR
  pbase = rbase * 128

  pltpu.sync_copy(ids_hbm.at[pl.ds(rbase, CR), :], idv)

  iot = _iota16()
  GR = 5

  def issue(g):
    for b in range(GR):
      j = g * GR + b
      pltpu.make_async_copy(ta_hbm.at[idv.at[j]], fpva.at[j], sa).start()
      pltpu.make_async_copy(tb_hbm.at[idv.at[j]], fpvb.at[j], sb).start()

  def drain():
    for b in range(GR):
      pltpu.make_async_copy(ta_hbm.at[idv.at[0]], fpva.at[0], sa).wait()
      pltpu.make_async_copy(tb_hbm.at[idv.at[0]], fpvb.at[0], sb).wait()

  def crow(r, carry):
    exs, tots = [], []
    for c in range(8):
      v = jnp.minimum(fpva[r, pl.ds(c * L, L)], fpvb[r, pl.ds(c * L, L)])
      fpva[r, pl.ds(c * L, L)] = v
      pos = (pbase + r * 128 + c * L) + iot
      isf = jnp.where(v == pos, 1, 0).astype(jnp.int32)
      exs.append(plsc.cumsum(isf) - isf)
      tots.append(jnp.sum(isf, dtype=jnp.int32))
    for c in range(8):
      sbuf[pl.ds(r * 128 + c * L, L)] = exs[c] + carry
      carry = carry + tots[c]
    return carry

  issue(jnp.int32(0))

  def grp(g, carry):
    g = g.astype(jnp.int32)

    @pl.when(g + 1 < CR // GR)
    def _():
      issue(g + 1)

    drain()
    for b in range(GR):
      carry = crow(g * GR + b, carry)
    return carry

  total = lax.fori_loop(0, CR // GR, grp, jnp.int32(0))

  pltpu.sync_copy(fpva, fpo_hbm.at[pl.ds(rbase, CR), :])
  pltpu.sync_copy(sbuf, s_hbm.at[pl.ds(pbase, CH)])
  pbuf[...] = jnp.full((L,), 0, jnp.int32) + total
  pltpu.sync_copy(pbuf, part_hbm.at[wid])


_phase_b = functools.partial(
    pl.kernel,
    out_type=(
        jax.ShapeDtypeStruct((ROWS, 128), jnp.int32),
        jax.ShapeDtypeStruct((N,), jnp.int32),
        jax.ShapeDtypeStruct((NW, L), jnp.int32),
    ),
    mesh=_mesh,
    compiler_params=_params,
    scratch_types=[
        pltpu.VMEM((CR, 128), jnp.int32),
        pltpu.VMEM((CR, 128), jnp.int32),
        pltpu.VMEM((CR, 128), jnp.int32),
        pltpu.VMEM((CH,), jnp.int32),
        pltpu.VMEM((L,), jnp.int32),
        pltpu.SemaphoreType.DMA,
        pltpu.SemaphoreType.DMA,
    ],
)(_kb)


def _kc(fpo_hbm, s_hbm, part_hbm, emb_hbm, out_hbm,
        pv, offv, fpv, sv, rb0, rb1, rb2, rb3,
        sem, sg0, sg1, sg2, sg3, sw0, sw1, sw2, sw3):
  wid = _wid()
  rbase = wid * CR

  pltpu.sync_copy(part_hbm, pv)
  iot = _iota16()
  z = jnp.zeros((L,), jnp.int32)
  t0 = plsc.load_gather(pv, [iot, z])
  t1 = plsc.load_gather(pv, [iot + 16, z])
  c0 = plsc.cumsum(t0)
  c1 = plsc.cumsum(t1) + jnp.sum(t0, dtype=jnp.int32)
  offv[pl.ds(0, L)] = c0 - t0
  offv[pl.ds(L, L)] = c1 - t1

  pltpu.sync_copy(fpo_hbm.at[pl.ds(rbase, CR), :], fpv)

  def ggrp(g, _):
    g = g.astype(jnp.int32)
    for b in range(10):
      j = g * 10 + b
      pltpu.make_async_copy(s_hbm.at[fpv.at[j]], sv.at[j], sem).start()
    for b in range(10):
      pltpu.make_async_copy(s_hbm.at[fpv.at[0]], sv.at[0], sem).wait()
    return 0

  lax.fori_loop(0, CR // 10, ggrp, 0)

  def hrow(r, _):
    r = r.astype(jnp.int32)
    for c in range(8):
      f = fpv[r, pl.ds(c * L, L)]
      s = sv[r, pl.ds(c * L, L)]
      o = plsc.load_gather(offv, [f // CH])
      sv[r, pl.ds(c * L, L)] = s + o
    return 0

  lax.fori_loop(0, CR, hrow, 0)

  obase = wid * CH
  rbs = (rb0, rb1, rb2, rb3)
  sgs = (sg0, sg1, sg2, sg3)
  sws = (sw0, sw1, sw2, sw3)

  def gst(j, b):
    pltpu.make_async_copy(emb_hbm.at[sv.at[j]], rbs[b], sgs[b]).start()

  def gwt(b):
    pltpu.make_async_copy(emb_hbm.at[sv.at[0]], rbs[b], sgs[b]).wait()

  def wst(j, b):
    pltpu.make_async_copy(
        rbs[b], out_hbm.at[pl.ds(obase + j * 128, 128), :], sws[b]).start()

  def wwt(b):
    pltpu.make_async_copy(
        rbs[b], out_hbm.at[pl.ds(obase, 128), :], sws[b]).wait()

  for b in range(4):
    gst(b, b)

  def grow(g, _):
    j0 = 4 * g.astype(jnp.int32)
    for b in range(4):
      j = j0 + b

      @pl.when(j < CR)
      def _():
        gwt(b)
        wst(j, b)

    for b in range(4):
      jn = j0 + 4 + b

      @pl.when(jn < CR)
      def _():
        wwt(b)
        gst(jn, b)

    return 0

  lax.fori_loop(0, (CR + 3) // 4, grow, 0)

  for b in range(4):
    wwt(b)


_phase_c = functools.partial(
    pl.kernel,
    out_type=jax.ShapeDtypeStruct((N, DIM), jnp.float32),
    mesh=_mesh,
    compiler_params=_params,
    scratch_types=[
        pltpu.VMEM((NW, L), jnp.int32),
        pltpu.VMEM((NW,), jnp.int32),
        pltpu.VMEM((CR, 128), jnp.int32),
        pltpu.VMEM((CR, 128), jnp.int32),
        pltpu.VMEM((128, DIM), jnp.float32),
        pltpu.VMEM((128, DIM), jnp.float32),
        pltpu.VMEM((128, DIM), jnp.float32),
        pltpu.VMEM((128, DIM), jnp.float32),
        pltpu.SemaphoreType.DMA,
        pltpu.SemaphoreType.DMA,
        pltpu.SemaphoreType.DMA,
        pltpu.SemaphoreType.DMA,
        pltpu.SemaphoreType.DMA,
        pltpu.SemaphoreType.DMA,
        pltpu.SemaphoreType.DMA,
        pltpu.SemaphoreType.DMA,
        pltpu.SemaphoreType.DMA,
    ],
)(_kc)


def kernel(ids, embedding_var, default_embedding):
  del default_embedding
  with jax.enable_x64(False):
    ids32 = ids.astype(jnp.int32).reshape(ROWS, 128)
    emb = embedding_var[:N]
    ta, tb = _phase_a(ids32)
    fpo, s_local, partials = _phase_b(ids32, ta, tb)
    out = _phase_c(fpo, s_local, partials, emb)
  return out.reshape(ids.shape + (DIM,))
